# baseline (speedup 1.0000x reference)
import functools

import jax
import jax.numpy as jnp
from jax import lax
from jax.experimental import pallas as pl
from jax.experimental.pallas import tpu as pltpu
from jax.experimental.pallas import tpu_sc as plsc

B, C, T = 128, 16, 16384
M = 2
NC, NS, L = 2, 16, 16
NW = NC * NS
SPB = B // NW
WINW = 384


def _mask_body(x_hbm, w_hbm, s_hbm, out_hbm, s_v, w_v, win_v):
    wid = lax.axis_index("s") * NC + lax.axis_index("c")
    b0 = wid * SPB

    pltpu.sync_copy(x_hbm.at[pl.ds(b0, SPB)], out_hbm.at[pl.ds(b0, SPB)])

    pltpu.sync_copy(s_hbm, s_v)
    pltpu.sync_copy(w_hbm, w_v)

    lanes = lax.broadcasted_iota(jnp.int32, (L,), 0)

    for j in range(SPB):
        b = b0 + j
        for m in range(M):
            idx = jnp.full((L,), m * B, dtype=jnp.int32) + b
            svec = plsc.load_gather(s_v, [idx])
            wvec = plsc.load_gather(w_v, [idx])
            evec = jnp.minimum(svec + wvec, T)
            s = svec[0]
            win = pl.multiple_of(
                jnp.minimum((s // 128) * 128, T - WINW), 128)

            pltpu.sync_copy(out_hbm.at[b, :, pl.ds(win, WINW)], win_v)

            keeps = []
            for t in range(WINW // L):
                p = win + t * L + lanes
                keeps.append((p < svec) | (p >= evec))

            def body(c, _):
                for t in range(WINW // L):
                    vec = win_v[c, pl.ds(t * L, L)]
                    win_v[c, pl.ds(t * L, L)] = jnp.where(
                        keeps[t], vec, 0.0)
                return 0

            lax.fori_loop(0, C, body, 0)
            pltpu.sync_copy(win_v, out_hbm.at[b, :, pl.ds(win, WINW)])


def kernel(x, widths, starts):
    mesh = plsc.VectorSubcoreMesh(
        core_axis_name="c", subcore_axis_name="s",
        num_cores=NC, num_subcores=NS)
    f = functools.partial(
        pl.kernel,
        out_type=jax.ShapeDtypeStruct((B, C, T), jnp.float32),
        mesh=mesh,
        compiler_params=pltpu.CompilerParams(needs_layout_passes=False),
        scratch_types=[
            pltpu.VMEM((M * B,), jnp.int32),
            pltpu.VMEM((M * B,), jnp.int32),
            pltpu.VMEM((C, WINW), jnp.float32),
        ],
    )(_mask_body)
    return f(x, widths.reshape(M * B), starts.reshape(M * B))


# speedup vs baseline: 48.6049x; 48.6049x over previous
import jax
import jax.numpy as jnp
from jax import lax
from jax.experimental import pallas as pl
from jax.experimental.pallas import tpu as pltpu

B, C, T = 128, 16, 16384
M = 2
BS = 4
WINW = 384


def _tc_body(w_ref, s_ref, x_ref, o_ref):
    g = pl.program_id(0)
    o_ref[...] = x_ref[...]
    pos = lax.broadcasted_iota(jnp.int32, (C, WINW), 1)
    for j in range(BS):
        b = g * BS + j
        for m in range(M):
            s = s_ref[m, b]
            e = jnp.minimum(s + w_ref[m, b], T)
            win = pl.multiple_of(
                jnp.minimum((s // 128) * 128, T - WINW), 128)
            p = pos + win
            keep = (p < s) | (p >= e)
            chunk = o_ref[j, :, pl.ds(win, WINW)]
            o_ref[j, :, pl.ds(win, WINW)] = jnp.where(keep, chunk, 0.0)


def kernel(x, widths, starts):
    return pl.pallas_call(
        _tc_body,
        out_shape=jax.ShapeDtypeStruct((B, C, T), jnp.float32),
        grid=(B // BS,),
        in_specs=[
            pl.BlockSpec(memory_space=pltpu.SMEM),
            pl.BlockSpec(memory_space=pltpu.SMEM),
            pl.BlockSpec((BS, C, T), lambda g: (g, 0, 0)),
        ],
        out_specs=pl.BlockSpec((BS, C, T), lambda g: (g, 0, 0)),
    )(widths, starts, x)


# speedup vs baseline: 49.6671x; 1.0219x over previous
import jax
import jax.numpy as jnp
from jax import lax
from jax.experimental import pallas as pl
from jax.experimental.pallas import tpu as pltpu

B, C, T = 128, 16, 16384
M = 2
BS = 8
WINW = 384


def _tc_body(w_ref, s_ref, x_ref, o_ref):
    g = pl.program_id(0)
    o_ref[...] = x_ref[...]
    pos = lax.broadcasted_iota(jnp.int32, (C, WINW), 1)
    for j in range(BS):
        b = g * BS + j
        for m in range(M):
            s = s_ref[m, b]
            e = jnp.minimum(s + w_ref[m, b], T)
            win = pl.multiple_of(
                jnp.minimum((s // 128) * 128, T - WINW), 128)
            p = pos + win
            keep = (p < s) | (p >= e)
            chunk = o_ref[j, :, pl.ds(win, WINW)]
            o_ref[j, :, pl.ds(win, WINW)] = jnp.where(keep, chunk, 0.0)


def kernel(x, widths, starts):
    return pl.pallas_call(
        _tc_body,
        out_shape=jax.ShapeDtypeStruct((B, C, T), jnp.float32),
        grid=(B // BS,),
        in_specs=[
            pl.BlockSpec(memory_space=pltpu.SMEM),
            pl.BlockSpec(memory_space=pltpu.SMEM),
            pl.BlockSpec((BS, C, T), lambda g: (g, 0, 0)),
        ],
        out_specs=pl.BlockSpec((BS, C, T), lambda g: (g, 0, 0)),
    )(widths, starts, x)
